# baseline (speedup 1.0000x reference)
import functools

import jax
import jax.numpy as jnp
from jax import lax
from jax.experimental import pallas as pl
from jax.experimental.pallas import tpu as pltpu
from jax.experimental.pallas import tpu_sc as plsc

N = 10000
E = 320000
D = 128
H = 128
K = 3

NC = 2
NS = 16
CHUNK = 128
CH0 = 121
CH1 = 36
CH = max(CH0, CH1)
NPAD = 10112
RPT = NPAD // NS
ZFULL = RPT // CHUNK
ZREM = RPT % CHUNK

_mesh = plsc.VectorSubcoreMesh(core_axis_name="c", subcore_axis_name="s")


@functools.partial(
    pl.kernel,
    out_type=jax.ShapeDtypeStruct((NC, NPAD, D), jnp.float32),
    mesh=_mesh,
    scratch_types=[
        pltpu.VMEM((CH, CHUNK), jnp.int32),
        pltpu.VMEM((CHUNK, D), jnp.float32),
        pltpu.VMEM((CHUNK, D), jnp.float32),
        pltpu.VMEM_SHARED((NPAD, D), jnp.float32),
        pltpu.SemaphoreType.DMA,
    ],
)
def _deg(dstr, ones_hbm, zeros_hbm, out_hbm, dst_v, ones_v, zr_v, acc_sh, sem):
    c = lax.axis_index("c")
    s = lax.axis_index("s")
    pltpu.sync_copy(dstr.at[c, s], dst_v)
    pltpu.sync_copy(ones_hbm, ones_v)
    pltpu.sync_copy(zeros_hbm, zr_v)
    base = s * RPT
    for k in range(ZFULL):
        pltpu.sync_copy(zr_v, acc_sh.at[pl.ds(base + CHUNK * k, CHUNK)])
    pltpu.sync_copy(zr_v.at[pl.ds(0, ZREM)],
                    acc_sh.at[pl.ds(base + ZFULL * CHUNK, ZREM)])
    plsc.subcore_barrier()

    def step(j, carry):
        pltpu.sync_copy(ones_v, acc_sh.at[dst_v.at[j]], add=True)
        return carry

    lax.fori_loop(0, jnp.where(c == 0, CH0, CH1), step, 0)
    plsc.subcore_barrier()
    for k in range(ZFULL):
        pltpu.sync_copy(acc_sh.at[pl.ds(base + CHUNK * k, CHUNK)], ones_v)
        pltpu.sync_copy(ones_v, out_hbm.at[c, pl.ds(base + CHUNK * k, CHUNK)])
    pltpu.sync_copy(acc_sh.at[pl.ds(base + ZFULL * CHUNK, ZREM)],
                    ones_v.at[pl.ds(0, ZREM)])
    pltpu.sync_copy(ones_v.at[pl.ds(0, ZREM)],
                    out_hbm.at[c, pl.ds(base + ZFULL * CHUNK, ZREM)])


@functools.partial(
    pl.kernel,
    out_type=jax.ShapeDtypeStruct((NC, NPAD, D), jnp.float32),
    mesh=_mesh,
    scratch_types=[
        pltpu.VMEM((CH, CHUNK), jnp.int32),
        pltpu.VMEM((CH, CHUNK), jnp.int32),
        pltpu.VMEM((CHUNK, D), jnp.float32),
        pltpu.VMEM_SHARED((NPAD, D), jnp.float32),
        pltpu.SemaphoreType.DMA,
    ],
)
def _spmm(g_hbm, srcr, dstr, zeros_hbm, out_hbm, src_v, dst_v, rows_v, acc_sh,
          sem):
    c = lax.axis_index("c")
    s = lax.axis_index("s")
    pltpu.sync_copy(srcr.at[c, s], src_v)
    pltpu.sync_copy(dstr.at[c, s], dst_v)
    pltpu.sync_copy(zeros_hbm, rows_v)
    base = s * RPT
    for k in range(ZFULL):
        pltpu.sync_copy(rows_v, acc_sh.at[pl.ds(base + CHUNK * k, CHUNK)])
    pltpu.sync_copy(rows_v.at[pl.ds(0, ZREM)],
                    acc_sh.at[pl.ds(base + ZFULL * CHUNK, ZREM)])
    plsc.subcore_barrier()

    def step(j, carry):
        pltpu.async_copy(g_hbm.at[src_v.at[j]], rows_v, sem).wait()
        pltpu.sync_copy(rows_v, acc_sh.at[dst_v.at[j]], add=True)
        return carry

    lax.fori_loop(0, jnp.where(c == 0, CH0, CH1), step, 0)
    plsc.subcore_barrier()
    for k in range(ZFULL):
        pltpu.sync_copy(acc_sh.at[pl.ds(base + CHUNK * k, CHUNK)], rows_v)
        pltpu.sync_copy(rows_v, out_hbm.at[c, pl.ds(base + CHUNK * k, CHUNK)])
    pltpu.sync_copy(acc_sh.at[pl.ds(base + ZFULL * CHUNK, ZREM)],
                    rows_v.at[pl.ds(0, ZREM)])
    pltpu.sync_copy(rows_v.at[pl.ds(0, ZREM)],
                    out_hbm.at[c, pl.ds(base + ZFULL * CHUNK, ZREM)])


BLK = 1000
GRID = N // BLK


def _mlp_ln_relu(h, W, b, ga, be):
    z = jnp.dot(h, W, preferred_element_type=jnp.float32) + b
    m = jnp.mean(z, axis=-1, keepdims=True)
    v = jnp.mean((z - m) * (z - m), axis=-1, keepdims=True)
    zn = (z - m) * lax.rsqrt(v + 1e-5) * ga + be
    return jnp.maximum(zn, 0.0)


def _row_spec():
    return pl.BlockSpec((BLK, D), lambda i: (i, 0))


def _full_spec(shape):
    return pl.BlockSpec(shape, lambda i: tuple(0 for _ in shape))


def _hop0_body(x_ref, d0_ref, d1_ref, W_ref, b_ref, ga_ref, be_ref,
               z_ref, g_ref, dinv_ref):
    deg = 1.0 + d0_ref[:, 0:1] + d1_ref[:, 0:1]
    dinv = lax.rsqrt(deg)
    x = x_ref[...]
    z_ref[...] = _mlp_ln_relu(x, W_ref[...], b_ref[...], ga_ref[...],
                              be_ref[...])
    g_ref[...] = x * dinv
    dinv_ref[...] = dinv


_hop0 = pl.pallas_call(
    _hop0_body,
    grid=(GRID,),
    in_specs=[
        _row_spec(),
        pl.BlockSpec((BLK, 16), lambda i: (i, 0)),
        pl.BlockSpec((BLK, 16), lambda i: (i, 0)),
        _full_spec((D, H)),
        _full_spec((1, H)),
        _full_spec((1, H)),
        _full_spec((1, H)),
    ],
    out_specs=[
        _row_spec(),
        _row_spec(),
        pl.BlockSpec((BLK, 1), lambda i: (i, 0)),
    ],
    out_shape=[
        jax.ShapeDtypeStruct((N, H), jnp.float32),
        jax.ShapeDtypeStruct((N, D), jnp.float32),
        jax.ShapeDtypeStruct((N, 1), jnp.float32),
    ],
)


def _hop_body(p0_ref, p1_ref, gp_ref, dinv_ref, W_ref, b_ref, ga_ref, be_ref,
              z_ref, g_ref):
    dinv = dinv_ref[...]
    h = (p0_ref[...] + p1_ref[...] + gp_ref[...]) * dinv
    z_ref[...] = _mlp_ln_relu(h, W_ref[...], b_ref[...], ga_ref[...],
                              be_ref[...])
    g_ref[...] = h * dinv


_hop = pl.pallas_call(
    _hop_body,
    grid=(GRID,),
    in_specs=[
        _row_spec(),
        _row_spec(),
        _row_spec(),
        pl.BlockSpec((BLK, 1), lambda i: (i, 0)),
        _full_spec((D, H)),
        _full_spec((1, H)),
        _full_spec((1, H)),
        _full_spec((1, H)),
    ],
    out_specs=[_row_spec(), _row_spec()],
    out_shape=[
        jax.ShapeDtypeStruct((N, H), jnp.float32),
        jax.ShapeDtypeStruct((N, D), jnp.float32),
    ],
)


def _final_body(z0_ref, z1_ref, z2_ref, z3_ref, Wr_ref, br_ref, gr_ref,
                ber_ref, out_ref):
    Wr = Wr_ref[...]
    acc = jnp.dot(z0_ref[...], Wr[0:H], preferred_element_type=jnp.float32)
    acc += jnp.dot(z1_ref[...], Wr[H:2 * H], preferred_element_type=jnp.float32)
    acc += jnp.dot(z2_ref[...], Wr[2 * H:3 * H],
                   preferred_element_type=jnp.float32)
    acc += jnp.dot(z3_ref[...], Wr[3 * H:4 * H],
                   preferred_element_type=jnp.float32)
    z = acc + br_ref[...]
    m = jnp.mean(z, axis=-1, keepdims=True)
    v = jnp.mean((z - m) * (z - m), axis=-1, keepdims=True)
    zn = (z - m) * lax.rsqrt(v + 1e-5) * gr_ref[...] + ber_ref[...]
    out_ref[...] = jnp.maximum(zn, 0.0)


_final = pl.pallas_call(
    _final_body,
    grid=(GRID,),
    in_specs=[
        _row_spec(),
        _row_spec(),
        _row_spec(),
        _row_spec(),
        _full_spec(((K + 1) * H, H)),
        _full_spec((1, H)),
        _full_spec((1, H)),
        _full_spec((1, H)),
    ],
    out_specs=_row_spec(),
    out_shape=jax.ShapeDtypeStruct((N, H), jnp.float32),
)


def kernel(x, edge_index, Ws, bs, gammas, betas, Wr, br, gr, ber):
    e0 = NS * CH0 * CHUNK
    pad1 = NS * CH1 * CHUNK - (E - e0)
    pad_dst = N + jnp.arange(pad1, dtype=jnp.int32) % (NPAD - N)
    s0 = jnp.concatenate(
        [edge_index[0, :e0].reshape(NS, CH0, CHUNK),
         jnp.zeros((NS, CH - CH0, CHUNK), jnp.int32)], axis=1)
    d0 = jnp.concatenate(
        [edge_index[1, :e0].reshape(NS, CH0, CHUNK),
         jnp.full((NS, CH - CH0, CHUNK), N, jnp.int32)], axis=1)
    s1 = jnp.concatenate(
        [edge_index[0, e0:], jnp.zeros((pad1,), jnp.int32)]).reshape(
            NS, CH1, CHUNK)
    d1 = jnp.concatenate(
        [edge_index[1, e0:], pad_dst]).reshape(NS, CH1, CHUNK)
    srcp = jnp.stack([s0, jnp.concatenate(
        [s1, jnp.zeros((NS, CH - CH1, CHUNK), jnp.int32)], axis=1)])
    dstp = jnp.stack([d0, jnp.concatenate(
        [d1, jnp.full((NS, CH - CH1, CHUNK), N, jnp.int32)], axis=1)])

    onesD = jnp.ones((CHUNK, D), jnp.float32)
    zerosD = jnp.zeros((CHUNK, D), jnp.float32)

    d = _deg(dstp, onesD, zerosD)
    b2 = bs.reshape(K + 1, 1, H)
    ga2 = gammas.reshape(K + 1, 1, H)
    be2 = betas.reshape(K + 1, 1, H)

    z0, g, dinv = _hop0(x, d[0, :N, :16], d[1, :N, :16], Ws[0], b2[0], ga2[0],
                        be2[0])
    zs = [z0]
    for i in range(1, K + 1):
        p = _spmm(g, srcp, dstp, zerosD)
        z, g = _hop(p[0, :N], p[1, :N], g, dinv, Ws[i], b2[i], ga2[i], be2[i])
        zs.append(z)

    return _final(zs[0], zs[1], zs[2], zs[3], Wr, br.reshape(1, H),
                  gr.reshape(1, H), ber.reshape(1, H))


# speedup vs baseline: 1.0852x; 1.0852x over previous
import functools

import jax
import jax.numpy as jnp
from jax import lax
from jax.experimental import pallas as pl
from jax.experimental.pallas import tpu as pltpu
from jax.experimental.pallas import tpu_sc as plsc

N = 10000
E = 320000
D = 128
H = 128
K = 3

NC = 2
NS = 16
CHUNK = 128
CH0 = 108
CH1 = 49
CH = max(CH0, CH1)
NPAD = 10112
RPT = NPAD // NS
ZFULL = RPT // CHUNK
ZREM = RPT % CHUNK

_mesh = plsc.VectorSubcoreMesh(core_axis_name="c", subcore_axis_name="s")


@functools.partial(
    pl.kernel,
    out_type=jax.ShapeDtypeStruct((NC, NPAD, D), jnp.float32),
    mesh=_mesh,
    scratch_types=[
        pltpu.VMEM((CH, CHUNK), jnp.int32),
        pltpu.VMEM((CHUNK, D), jnp.float32),
        pltpu.VMEM((CHUNK, D), jnp.float32),
        pltpu.VMEM_SHARED((NPAD, D), jnp.float32),
        pltpu.SemaphoreType.DMA,
    ],
)
def _deg(dstr, ones_hbm, zeros_hbm, out_hbm, dst_v, ones_v, zr_v, acc_sh, sem):
    c = lax.axis_index("c")
    s = lax.axis_index("s")
    pltpu.sync_copy(dstr.at[c, s], dst_v)
    pltpu.sync_copy(ones_hbm, ones_v)
    pltpu.sync_copy(zeros_hbm, zr_v)
    base = s * RPT
    for k in range(ZFULL):
        pltpu.sync_copy(zr_v, acc_sh.at[pl.ds(base + CHUNK * k, CHUNK)])
    pltpu.sync_copy(zr_v.at[pl.ds(0, ZREM)],
                    acc_sh.at[pl.ds(base + ZFULL * CHUNK, ZREM)])
    plsc.subcore_barrier()

    def step(j, carry):
        pltpu.sync_copy(ones_v, acc_sh.at[dst_v.at[j]], add=True)
        return carry

    lax.fori_loop(0, jnp.where(c == 0, CH0, CH1), step, 0)
    plsc.subcore_barrier()
    for k in range(ZFULL):
        pltpu.sync_copy(acc_sh.at[pl.ds(base + CHUNK * k, CHUNK)], ones_v)
        pltpu.sync_copy(ones_v, out_hbm.at[c, pl.ds(base + CHUNK * k, CHUNK)])
    pltpu.sync_copy(acc_sh.at[pl.ds(base + ZFULL * CHUNK, ZREM)],
                    ones_v.at[pl.ds(0, ZREM)])
    pltpu.sync_copy(ones_v.at[pl.ds(0, ZREM)],
                    out_hbm.at[c, pl.ds(base + ZFULL * CHUNK, ZREM)])


@functools.partial(
    pl.kernel,
    out_type=jax.ShapeDtypeStruct((NC, NPAD, D), jnp.float32),
    mesh=_mesh,
    scratch_types=[
        pltpu.VMEM((CH, CHUNK), jnp.int32),
        pltpu.VMEM((CH, CHUNK), jnp.int32),
        pltpu.VMEM((CHUNK, D), jnp.float32),
        pltpu.VMEM_SHARED((NPAD, D), jnp.float32),
        pltpu.SemaphoreType.DMA,
    ],
)
def _spmm(g_hbm, srcr, dstr, zeros_hbm, out_hbm, src_v, dst_v, rows_v, acc_sh,
          sem):
    c = lax.axis_index("c")
    s = lax.axis_index("s")
    pltpu.sync_copy(srcr.at[c, s], src_v)
    pltpu.sync_copy(dstr.at[c, s], dst_v)
    pltpu.sync_copy(zeros_hbm, rows_v)
    base = s * RPT
    for k in range(ZFULL):
        pltpu.sync_copy(rows_v, acc_sh.at[pl.ds(base + CHUNK * k, CHUNK)])
    pltpu.sync_copy(rows_v.at[pl.ds(0, ZREM)],
                    acc_sh.at[pl.ds(base + ZFULL * CHUNK, ZREM)])
    plsc.subcore_barrier()

    def step(j, carry):
        pltpu.async_copy(g_hbm.at[src_v.at[j]], rows_v, sem).wait()
        pltpu.sync_copy(rows_v, acc_sh.at[dst_v.at[j]], add=True)
        return carry

    lax.fori_loop(0, jnp.where(c == 0, CH0, CH1), step, 0)
    plsc.subcore_barrier()
    for k in range(ZFULL):
        pltpu.sync_copy(acc_sh.at[pl.ds(base + CHUNK * k, CHUNK)], rows_v)
        pltpu.sync_copy(rows_v, out_hbm.at[c, pl.ds(base + CHUNK * k, CHUNK)])
    pltpu.sync_copy(acc_sh.at[pl.ds(base + ZFULL * CHUNK, ZREM)],
                    rows_v.at[pl.ds(0, ZREM)])
    pltpu.sync_copy(rows_v.at[pl.ds(0, ZREM)],
                    out_hbm.at[c, pl.ds(base + ZFULL * CHUNK, ZREM)])


BLK = 1000
GRID = N // BLK


def _mlp_ln_relu(h, W, b, ga, be):
    z = jnp.dot(h, W, preferred_element_type=jnp.float32) + b
    m = jnp.mean(z, axis=-1, keepdims=True)
    v = jnp.mean((z - m) * (z - m), axis=-1, keepdims=True)
    zn = (z - m) * lax.rsqrt(v + 1e-5) * ga + be
    return jnp.maximum(zn, 0.0)


def _row_spec():
    return pl.BlockSpec((BLK, D), lambda i: (i, 0))


def _full_spec(shape):
    return pl.BlockSpec(shape, lambda i: tuple(0 for _ in shape))


def _hop0_body(x_ref, d0_ref, d1_ref, W_ref, b_ref, ga_ref, be_ref,
               z_ref, g_ref, dinv_ref):
    deg = 1.0 + d0_ref[:, 0:1] + d1_ref[:, 0:1]
    dinv = lax.rsqrt(deg)
    x = x_ref[...]
    z_ref[...] = _mlp_ln_relu(x, W_ref[...], b_ref[...], ga_ref[...],
                              be_ref[...])
    g_ref[...] = x * dinv
    dinv_ref[...] = dinv


_hop0 = pl.pallas_call(
    _hop0_body,
    grid=(GRID,),
    in_specs=[
        _row_spec(),
        pl.BlockSpec((BLK, 16), lambda i: (i, 0)),
        pl.BlockSpec((BLK, 16), lambda i: (i, 0)),
        _full_spec((D, H)),
        _full_spec((1, H)),
        _full_spec((1, H)),
        _full_spec((1, H)),
    ],
    out_specs=[
        _row_spec(),
        _row_spec(),
        pl.BlockSpec((BLK, 1), lambda i: (i, 0)),
    ],
    out_shape=[
        jax.ShapeDtypeStruct((N, H), jnp.float32),
        jax.ShapeDtypeStruct((N, D), jnp.float32),
        jax.ShapeDtypeStruct((N, 1), jnp.float32),
    ],
)


def _hop_body(p0_ref, p1_ref, gp_ref, dinv_ref, W_ref, b_ref, ga_ref, be_ref,
              z_ref, g_ref):
    dinv = dinv_ref[...]
    h = (p0_ref[...] + p1_ref[...] + gp_ref[...]) * dinv
    z_ref[...] = _mlp_ln_relu(h, W_ref[...], b_ref[...], ga_ref[...],
                              be_ref[...])
    g_ref[...] = h * dinv


_hop = pl.pallas_call(
    _hop_body,
    grid=(GRID,),
    in_specs=[
        _row_spec(),
        _row_spec(),
        _row_spec(),
        pl.BlockSpec((BLK, 1), lambda i: (i, 0)),
        _full_spec((D, H)),
        _full_spec((1, H)),
        _full_spec((1, H)),
        _full_spec((1, H)),
    ],
    out_specs=[_row_spec(), _row_spec()],
    out_shape=[
        jax.ShapeDtypeStruct((N, H), jnp.float32),
        jax.ShapeDtypeStruct((N, D), jnp.float32),
    ],
)


def _final_body(z0_ref, z1_ref, z2_ref, z3_ref, Wr_ref, br_ref, gr_ref,
                ber_ref, out_ref):
    Wr = Wr_ref[...]
    acc = jnp.dot(z0_ref[...], Wr[0:H], preferred_element_type=jnp.float32)
    acc += jnp.dot(z1_ref[...], Wr[H:2 * H], preferred_element_type=jnp.float32)
    acc += jnp.dot(z2_ref[...], Wr[2 * H:3 * H],
                   preferred_element_type=jnp.float32)
    acc += jnp.dot(z3_ref[...], Wr[3 * H:4 * H],
                   preferred_element_type=jnp.float32)
    z = acc + br_ref[...]
    m = jnp.mean(z, axis=-1, keepdims=True)
    v = jnp.mean((z - m) * (z - m), axis=-1, keepdims=True)
    zn = (z - m) * lax.rsqrt(v + 1e-5) * gr_ref[...] + ber_ref[...]
    out_ref[...] = jnp.maximum(zn, 0.0)


_final = pl.pallas_call(
    _final_body,
    grid=(GRID,),
    in_specs=[
        _row_spec(),
        _row_spec(),
        _row_spec(),
        _row_spec(),
        _full_spec(((K + 1) * H, H)),
        _full_spec((1, H)),
        _full_spec((1, H)),
        _full_spec((1, H)),
    ],
    out_specs=_row_spec(),
    out_shape=jax.ShapeDtypeStruct((N, H), jnp.float32),
)


def kernel(x, edge_index, Ws, bs, gammas, betas, Wr, br, gr, ber):
    e0 = NS * CH0 * CHUNK
    pad1 = NS * CH1 * CHUNK - (E - e0)
    pad_dst = N + jnp.arange(pad1, dtype=jnp.int32) % (NPAD - N)
    s0 = jnp.concatenate(
        [edge_index[0, :e0].reshape(NS, CH0, CHUNK),
         jnp.zeros((NS, CH - CH0, CHUNK), jnp.int32)], axis=1)
    d0 = jnp.concatenate(
        [edge_index[1, :e0].reshape(NS, CH0, CHUNK),
         jnp.full((NS, CH - CH0, CHUNK), N, jnp.int32)], axis=1)
    s1 = jnp.concatenate(
        [edge_index[0, e0:], jnp.zeros((pad1,), jnp.int32)]).reshape(
            NS, CH1, CHUNK)
    d1 = jnp.concatenate(
        [edge_index[1, e0:], pad_dst]).reshape(NS, CH1, CHUNK)
    srcp = jnp.stack([s0, jnp.concatenate(
        [s1, jnp.zeros((NS, CH - CH1, CHUNK), jnp.int32)], axis=1)])
    dstp = jnp.stack([d0, jnp.concatenate(
        [d1, jnp.full((NS, CH - CH1, CHUNK), N, jnp.int32)], axis=1)])

    onesD = jnp.ones((CHUNK, D), jnp.float32)
    zerosD = jnp.zeros((CHUNK, D), jnp.float32)

    d = _deg(dstp, onesD, zerosD)
    b2 = bs.reshape(K + 1, 1, H)
    ga2 = gammas.reshape(K + 1, 1, H)
    be2 = betas.reshape(K + 1, 1, H)

    z0, g, dinv = _hop0(x, d[0, :N, :16], d[1, :N, :16], Ws[0], b2[0], ga2[0],
                        be2[0])
    zs = [z0]
    for i in range(1, K + 1):
        p = _spmm(g, srcp, dstp, zerosD)
        z, g = _hop(p[0, :N], p[1, :N], g, dinv, Ws[i], b2[i], ga2[i], be2[i])
        zs.append(z)

    return _final(zs[0], zs[1], zs[2], zs[3], Wr, br.reshape(1, H),
                  gr.reshape(1, H), ber.reshape(1, H))


# speedup vs baseline: 1.1270x; 1.0385x over previous
import functools

import jax
import jax.numpy as jnp
from jax import lax
from jax.experimental import pallas as pl
from jax.experimental.pallas import tpu as pltpu
from jax.experimental.pallas import tpu_sc as plsc

N = 10000
E = 320000
D = 128
H = 128
K = 3

NC = 2
NS = 16
CHUNK = 128
CH0 = 103
CH1 = 54
CH = max(CH0, CH1)
NPAD = 10112
RPT = NPAD // NS
ZFULL = RPT // CHUNK
ZREM = RPT % CHUNK

_mesh = plsc.VectorSubcoreMesh(core_axis_name="c", subcore_axis_name="s")


@functools.partial(
    pl.kernel,
    out_type=jax.ShapeDtypeStruct((NC, NPAD, D), jnp.float32),
    mesh=_mesh,
    scratch_types=[
        pltpu.VMEM((CH, CHUNK), jnp.int32),
        pltpu.VMEM((CHUNK, D), jnp.float32),
        pltpu.VMEM((CHUNK, D), jnp.float32),
        pltpu.VMEM_SHARED((NPAD, D), jnp.float32),
        pltpu.SemaphoreType.DMA,
    ],
)
def _deg(dstr, ones_hbm, zeros_hbm, out_hbm, dst_v, ones_v, zr_v, acc_sh, sem):
    c = lax.axis_index("c")
    s = lax.axis_index("s")
    pltpu.sync_copy(dstr.at[c, s], dst_v)
    pltpu.sync_copy(ones_hbm, ones_v)
    pltpu.sync_copy(zeros_hbm, zr_v)
    base = s * RPT
    for k in range(ZFULL):
        pltpu.sync_copy(zr_v, acc_sh.at[pl.ds(base + CHUNK * k, CHUNK)])
    pltpu.sync_copy(zr_v.at[pl.ds(0, ZREM)],
                    acc_sh.at[pl.ds(base + ZFULL * CHUNK, ZREM)])
    plsc.subcore_barrier()

    def step(j, carry):
        pltpu.sync_copy(ones_v, acc_sh.at[dst_v.at[j]], add=True)
        return carry

    lax.fori_loop(0, jnp.where(c == 0, CH0, CH1), step, 0)
    plsc.subcore_barrier()
    for k in range(ZFULL):
        pltpu.sync_copy(acc_sh.at[pl.ds(base + CHUNK * k, CHUNK)], ones_v)
        pltpu.sync_copy(ones_v, out_hbm.at[c, pl.ds(base + CHUNK * k, CHUNK)])
    pltpu.sync_copy(acc_sh.at[pl.ds(base + ZFULL * CHUNK, ZREM)],
                    ones_v.at[pl.ds(0, ZREM)])
    pltpu.sync_copy(ones_v.at[pl.ds(0, ZREM)],
                    out_hbm.at[c, pl.ds(base + ZFULL * CHUNK, ZREM)])


@functools.partial(
    pl.kernel,
    out_type=jax.ShapeDtypeStruct((NC, NPAD, D), jnp.float32),
    mesh=_mesh,
    scratch_types=[
        pltpu.VMEM((CH, CHUNK), jnp.int32),
        pltpu.VMEM((CH, CHUNK), jnp.int32),
        pltpu.VMEM((CHUNK, D), jnp.float32),
        pltpu.VMEM_SHARED((NPAD, D), jnp.float32),
        pltpu.SemaphoreType.DMA,
    ],
)
def _spmm(g_hbm, srcr, dstr, zeros_hbm, out_hbm, src_v, dst_v, rows_v, acc_sh,
          sem):
    c = lax.axis_index("c")
    s = lax.axis_index("s")
    pltpu.sync_copy(srcr.at[c, s], src_v)
    pltpu.sync_copy(dstr.at[c, s], dst_v)
    pltpu.sync_copy(zeros_hbm, rows_v)
    base = s * RPT
    for k in range(ZFULL):
        pltpu.sync_copy(rows_v, acc_sh.at[pl.ds(base + CHUNK * k, CHUNK)])
    pltpu.sync_copy(rows_v.at[pl.ds(0, ZREM)],
                    acc_sh.at[pl.ds(base + ZFULL * CHUNK, ZREM)])
    plsc.subcore_barrier()

    def step(j, carry):
        pltpu.async_copy(g_hbm.at[src_v.at[j]], rows_v, sem).wait()
        pltpu.sync_copy(rows_v, acc_sh.at[dst_v.at[j]], add=True)
        return carry

    lax.fori_loop(0, jnp.where(c == 0, CH0, CH1), step, 0)
    plsc.subcore_barrier()
    for k in range(ZFULL):
        pltpu.sync_copy(acc_sh.at[pl.ds(base + CHUNK * k, CHUNK)], rows_v)
        pltpu.sync_copy(rows_v, out_hbm.at[c, pl.ds(base + CHUNK * k, CHUNK)])
    pltpu.sync_copy(acc_sh.at[pl.ds(base + ZFULL * CHUNK, ZREM)],
                    rows_v.at[pl.ds(0, ZREM)])
    pltpu.sync_copy(rows_v.at[pl.ds(0, ZREM)],
                    out_hbm.at[c, pl.ds(base + ZFULL * CHUNK, ZREM)])


BLK = 1000
GRID = N // BLK


def _mlp_ln_relu(h, W, b, ga, be):
    z = jnp.dot(h, W, preferred_element_type=jnp.float32) + b
    m = jnp.mean(z, axis=-1, keepdims=True)
    v = jnp.mean((z - m) * (z - m), axis=-1, keepdims=True)
    zn = (z - m) * lax.rsqrt(v + 1e-5) * ga + be
    return jnp.maximum(zn, 0.0)


def _row_spec():
    return pl.BlockSpec((BLK, D), lambda i: (i, 0))


def _full_spec(shape):
    return pl.BlockSpec(shape, lambda i: tuple(0 for _ in shape))


def _hop0_body(x_ref, d0_ref, d1_ref, W_ref, b_ref, ga_ref, be_ref,
               z_ref, g_ref, dinv_ref):
    deg = 1.0 + d0_ref[:, 0:1] + d1_ref[:, 0:1]
    dinv = lax.rsqrt(deg)
    x = x_ref[...]
    z_ref[...] = _mlp_ln_relu(x, W_ref[...], b_ref[...], ga_ref[...],
                              be_ref[...])
    g_ref[...] = x * dinv
    dinv_ref[...] = dinv


_hop0 = pl.pallas_call(
    _hop0_body,
    grid=(GRID,),
    in_specs=[
        _row_spec(),
        pl.BlockSpec((BLK, 16), lambda i: (i, 0)),
        pl.BlockSpec((BLK, 16), lambda i: (i, 0)),
        _full_spec((D, H)),
        _full_spec((1, H)),
        _full_spec((1, H)),
        _full_spec((1, H)),
    ],
    out_specs=[
        _row_spec(),
        _row_spec(),
        pl.BlockSpec((BLK, 1), lambda i: (i, 0)),
    ],
    out_shape=[
        jax.ShapeDtypeStruct((N, H), jnp.float32),
        jax.ShapeDtypeStruct((N, D), jnp.float32),
        jax.ShapeDtypeStruct((N, 1), jnp.float32),
    ],
)


def _hop_body(p0_ref, p1_ref, gp_ref, dinv_ref, W_ref, b_ref, ga_ref, be_ref,
              z_ref, g_ref):
    dinv = dinv_ref[...]
    h = (p0_ref[...] + p1_ref[...] + gp_ref[...]) * dinv
    z_ref[...] = _mlp_ln_relu(h, W_ref[...], b_ref[...], ga_ref[...],
                              be_ref[...])
    g_ref[...] = h * dinv


_hop = pl.pallas_call(
    _hop_body,
    grid=(GRID,),
    in_specs=[
        _row_spec(),
        _row_spec(),
        _row_spec(),
        pl.BlockSpec((BLK, 1), lambda i: (i, 0)),
        _full_spec((D, H)),
        _full_spec((1, H)),
        _full_spec((1, H)),
        _full_spec((1, H)),
    ],
    out_specs=[_row_spec(), _row_spec()],
    out_shape=[
        jax.ShapeDtypeStruct((N, H), jnp.float32),
        jax.ShapeDtypeStruct((N, D), jnp.float32),
    ],
)


def _final_body(z0_ref, z1_ref, z2_ref, z3_ref, Wr_ref, br_ref, gr_ref,
                ber_ref, out_ref):
    Wr = Wr_ref[...]
    acc = jnp.dot(z0_ref[...], Wr[0:H], preferred_element_type=jnp.float32)
    acc += jnp.dot(z1_ref[...], Wr[H:2 * H], preferred_element_type=jnp.float32)
    acc += jnp.dot(z2_ref[...], Wr[2 * H:3 * H],
                   preferred_element_type=jnp.float32)
    acc += jnp.dot(z3_ref[...], Wr[3 * H:4 * H],
                   preferred_element_type=jnp.float32)
    z = acc + br_ref[...]
    m = jnp.mean(z, axis=-1, keepdims=True)
    v = jnp.mean((z - m) * (z - m), axis=-1, keepdims=True)
    zn = (z - m) * lax.rsqrt(v + 1e-5) * gr_ref[...] + ber_ref[...]
    out_ref[...] = jnp.maximum(zn, 0.0)


_final = pl.pallas_call(
    _final_body,
    grid=(GRID,),
    in_specs=[
        _row_spec(),
        _row_spec(),
        _row_spec(),
        _row_spec(),
        _full_spec(((K + 1) * H, H)),
        _full_spec((1, H)),
        _full_spec((1, H)),
        _full_spec((1, H)),
    ],
    out_specs=_row_spec(),
    out_shape=jax.ShapeDtypeStruct((N, H), jnp.float32),
)


def kernel(x, edge_index, Ws, bs, gammas, betas, Wr, br, gr, ber):
    e0 = NS * CH0 * CHUNK
    pad1 = NS * CH1 * CHUNK - (E - e0)
    pad_dst = N + jnp.arange(pad1, dtype=jnp.int32) % (NPAD - N)
    s0 = jnp.concatenate(
        [edge_index[0, :e0].reshape(NS, CH0, CHUNK),
         jnp.zeros((NS, CH - CH0, CHUNK), jnp.int32)], axis=1)
    d0 = jnp.concatenate(
        [edge_index[1, :e0].reshape(NS, CH0, CHUNK),
         jnp.full((NS, CH - CH0, CHUNK), N, jnp.int32)], axis=1)
    s1 = jnp.concatenate(
        [edge_index[0, e0:], jnp.zeros((pad1,), jnp.int32)]).reshape(
            NS, CH1, CHUNK)
    d1 = jnp.concatenate(
        [edge_index[1, e0:], pad_dst]).reshape(NS, CH1, CHUNK)
    srcp = jnp.stack([s0, jnp.concatenate(
        [s1, jnp.zeros((NS, CH - CH1, CHUNK), jnp.int32)], axis=1)])
    dstp = jnp.stack([d0, jnp.concatenate(
        [d1, jnp.full((NS, CH - CH1, CHUNK), N, jnp.int32)], axis=1)])

    onesD = jnp.ones((CHUNK, D), jnp.float32)
    zerosD = jnp.zeros((CHUNK, D), jnp.float32)

    d = _deg(dstp, onesD, zerosD)
    b2 = bs.reshape(K + 1, 1, H)
    ga2 = gammas.reshape(K + 1, 1, H)
    be2 = betas.reshape(K + 1, 1, H)

    z0, g, dinv = _hop0(x, d[0, :N, :16], d[1, :N, :16], Ws[0], b2[0], ga2[0],
                        be2[0])
    zs = [z0]
    for i in range(1, K + 1):
        p = _spmm(g, srcp, dstp, zerosD)
        z, g = _hop(p[0, :N], p[1, :N], g, dinv, Ws[i], b2[i], ga2[i], be2[i])
        zs.append(z)

    return _final(zs[0], zs[1], zs[2], zs[3], Wr, br.reshape(1, H),
                  gr.reshape(1, H), ber.reshape(1, H))
